# Initial kernel scaffold; baseline (speedup 1.0000x reference)
import jax
import jax.numpy as jnp
from jax.experimental import pallas as pl


def kernel(xyz, features, W1, g1, b1, W2, g2, b2):
    raise NotImplementedError("write your pallas kernel here")


# speedup vs baseline: 10.9556x; 10.9556x over previous
import functools

import jax
import jax.numpy as jnp
from jax import lax
from jax.experimental import pallas as pl
from jax.experimental.pallas import tpu as pltpu
from jax.experimental.pallas import tpu_sc as plsc

NPOINT = 2048
K = 32


def _fps_body(x_ref, y_ref, z_ref, far0_ref, cent_ref, nx_ref, ny_ref, nz_ref,
              dist_ref, *, npoint):
    B, N = x_ref.shape
    col = lax.broadcasted_iota(jnp.int32, (B, N), 1)
    dist_ref[...] = jnp.full((B, N), 1e10, jnp.float32)
    x = x_ref[...]
    y = y_ref[...]
    z = z_ref[...]

    blk = 128
    lane = lax.broadcasted_iota(jnp.int32, (B, blk), 1)

    def step(i, carry):
        far, hc, hx, hy, hz = carry
        sel = col == far
        cx = jnp.min(jnp.where(sel, x, jnp.float32(1e30)), axis=1, keepdims=True)
        cy = jnp.min(jnp.where(sel, y, jnp.float32(1e30)), axis=1, keepdims=True)
        cz = jnp.min(jnp.where(sel, z, jnp.float32(1e30)), axis=1, keepdims=True)
        m = lane == i
        mf = m.astype(jnp.float32)
        hc = hc + far * m.astype(jnp.int32)
        hx = hx + cx * mf
        hy = hy + cy * mf
        hz = hz + cz * mf
        dx = x - cx
        dy = y - cy
        dz = z - cz
        d = dx * dx + dy * dy + dz * dz
        nd = jnp.minimum(dist_ref[...], d)
        dist_ref[...] = nd
        mx = jnp.max(nd, axis=1, keepdims=True)
        nf = jnp.min(jnp.where(nd == mx, col, jnp.int32(N)), axis=1,
                     keepdims=True)
        return nf, hc, hx, hy, hz

    far = far0_ref[...]
    for b in range(npoint // blk):
        hc = jnp.zeros((B, blk), jnp.int32)
        hf = jnp.zeros((B, blk), jnp.float32)
        far, hc, hx, hy, hz = lax.fori_loop(0, blk, step,
                                            (far, hc, hf, hf, hf))
        cent_ref[:, b * blk:(b + 1) * blk] = hc
        nx_ref[:, b * blk:(b + 1) * blk] = hx
        ny_ref[:, b * blk:(b + 1) * blk] = hy
        nz_ref[:, b * blk:(b + 1) * blk] = hz


def _run_fps(xyz, npoint):
    B, N, _ = xyz.shape
    xt = jnp.transpose(xyz, (0, 2, 1))
    far0 = jax.random.randint(jax.random.key(1), (B,), 0, N)
    far0 = far0.astype(jnp.int32).reshape(B, 1)
    out_shapes = (
        jax.ShapeDtypeStruct((B, npoint), jnp.int32),
        jax.ShapeDtypeStruct((B, npoint), jnp.float32),
        jax.ShapeDtypeStruct((B, npoint), jnp.float32),
        jax.ShapeDtypeStruct((B, npoint), jnp.float32),
    )
    return pl.pallas_call(
        functools.partial(_fps_body, npoint=npoint),
        out_shape=out_shapes,
        scratch_shapes=[pltpu.VMEM((B, N), jnp.float32)],
    )(xt[:, 0], xt[:, 1], xt[:, 2], far0)


def _knn_body(nx_ref, ny_ref, nz_ref, x_ref, idx_ref, dist_ref, *, k):
    TS = nx_ref.shape[1]
    N = x_ref.shape[2]
    cx = nx_ref[0]
    cy = ny_ref[0]
    cz = nz_ref[0]
    pts = x_ref[0]
    px = pts[0:1]
    py = pts[1:2]
    pz = pts[2:3]
    s2 = cx * cx + cy * cy + cz * cz
    d2 = px * px + py * py + pz * pz
    q = jnp.concatenate([cx, cy, cz], axis=1)
    cross = jax.lax.dot_general(
        q.astype(jnp.bfloat16), pts.astype(jnp.bfloat16),
        (((1,), (0,)), ((), ())), preferred_element_type=jnp.float32)
    dist_ref[...] = jnp.maximum(s2 + d2 - 2.0 * cross, 0.0)
    col = lax.broadcasted_iota(jnp.int32, (TS, N), 1)
    kcol = lax.broadcasted_iota(jnp.int32, (TS, k), 1)

    def step(j, acc):
        d = dist_ref[...]
        m = jnp.min(d, axis=1, keepdims=True)
        a = jnp.min(jnp.where(d == m, col, jnp.int32(N)), axis=1,
                    keepdims=True)
        dist_ref[...] = jnp.where(col == a, jnp.float32(1e30), d)
        return acc + a * (kcol == j).astype(jnp.int32)

    idx_ref[0] = lax.fori_loop(0, k, step, jnp.zeros((TS, k), jnp.int32))


def _run_knn(nx, ny, nz, xyz, k, ts):
    B, S = nx.shape
    N = xyz.shape[1]
    xt = jnp.transpose(xyz, (0, 2, 1))
    nx3 = nx.reshape(B, S, 1)
    ny3 = ny.reshape(B, S, 1)
    nz3 = nz.reshape(B, S, 1)
    grid = (B, S // ts)
    q_spec = pl.BlockSpec((1, ts, 1), lambda b, s: (b, s, 0))
    return pl.pallas_call(
        functools.partial(_knn_body, k=k),
        grid=grid,
        in_specs=[q_spec, q_spec, q_spec,
                  pl.BlockSpec((1, 3, N), lambda b, s: (b, 0, 0))],
        out_specs=pl.BlockSpec((1, ts, k), lambda b, s: (b, s, 0)),
        out_shape=jax.ShapeDtypeStruct((B, S, k), jnp.int32),
        scratch_shapes=[pltpu.VMEM((ts, N), jnp.float32)],
    )(nx3, ny3, nz3, xt)


def _run_sc_gather(tbl, gidx):
    P = gidx.shape[0]
    D = tbl.shape[1]
    info = plsc.get_sparse_core_info()
    NC, NS = info.num_cores, info.num_subcores
    NW = NC * NS
    per_w = P // NW
    CH = 128
    n_chunks = per_w // CH
    mesh = plsc.VectorSubcoreMesh(core_axis_name="c", subcore_axis_name="s")

    @functools.partial(
        pl.kernel, mesh=mesh,
        out_type=jax.ShapeDtypeStruct((P, D), jnp.float32),
        scratch_types=[
            pltpu.VMEM((CH,), jnp.int32),
            pltpu.VMEM((CH, D), jnp.float32),
            pltpu.SemaphoreType.DMA,
        ],
    )
    def gather_k(tbl_hbm, idx_hbm, out_hbm, idx_v, rows_v, sem):
        wid = lax.axis_index("s") * NC + lax.axis_index("c")

        def chunk(c, _):
            base = wid * per_w + c * CH
            pltpu.sync_copy(idx_hbm.at[pl.ds(base, CH)], idx_v)
            pltpu.async_copy(tbl_hbm.at[idx_v], rows_v, sem).wait()
            pltpu.sync_copy(rows_v, out_hbm.at[pl.ds(base, CH)])
            return 0

        lax.fori_loop(0, n_chunks, chunk, 0)

    return gather_k(tbl, gidx)


def _mlp_stats1_body(gr_ref, nx_ref, w1pt_ref, w1at_ref, st_ref, *, k):
    t = pl.program_id(0)
    g = gr_ref[...]
    TG = nx_ref.shape[0]
    y1 = jnp.dot(g, w1pt_ref[...], preferred_element_type=jnp.float32)
    nw = jnp.dot(nx_ref[...], w1at_ref[...],
                 preferred_element_type=jnp.float32)
    C1 = y1.shape[1]
    nwr = jnp.broadcast_to(nw[:, None, :], (TG, k, C1)).reshape(TG * k, C1)
    y1 = y1 - nwr
    s = jnp.sum(y1, axis=0, keepdims=True)
    q = jnp.sum(y1 * y1, axis=0, keepdims=True)
    contrib = jnp.concatenate(
        [s, q, jnp.zeros((6, C1), jnp.float32)], axis=0)

    @pl.when(t == 0)
    def _():
        st_ref[...] = jnp.zeros_like(st_ref)

    st_ref[...] += contrib


def _mlp_stats2_body(gr_ref, nx_ref, w1pt_ref, w1at_ref, sc1_ref, w2t_ref,
                     st_ref, *, k):
    t = pl.program_id(0)
    g = gr_ref[...]
    TG = nx_ref.shape[0]
    y1 = jnp.dot(g, w1pt_ref[...], preferred_element_type=jnp.float32)
    nw = jnp.dot(nx_ref[...], w1at_ref[...],
                 preferred_element_type=jnp.float32)
    C1 = y1.shape[1]
    nwr = jnp.broadcast_to(nw[:, None, :], (TG, k, C1)).reshape(TG * k, C1)
    h1 = jnp.maximum((y1 - nwr) * sc1_ref[0:1, :] + sc1_ref[1:2, :], 0.0)
    y2 = jnp.dot(h1, w2t_ref[...], preferred_element_type=jnp.float32)
    C2 = y2.shape[1]
    s = jnp.sum(y2, axis=0, keepdims=True)
    q = jnp.sum(y2 * y2, axis=0, keepdims=True)
    contrib = jnp.concatenate(
        [s, q, jnp.zeros((6, C2), jnp.float32)], axis=0)

    @pl.when(t == 0)
    def _():
        st_ref[...] = jnp.zeros_like(st_ref)

    st_ref[...] += contrib


def _mlp_final_body(gr_ref, nx_ref, w1pt_ref, w1at_ref, sc1_ref, w2t_ref,
                    sc2_ref, out_ref, *, k):
    g = gr_ref[...]
    TG = nx_ref.shape[0]
    y1 = jnp.dot(g, w1pt_ref[...], preferred_element_type=jnp.float32)
    nw = jnp.dot(nx_ref[...], w1at_ref[...],
                 preferred_element_type=jnp.float32)
    C1 = y1.shape[1]
    nwr = jnp.broadcast_to(nw[:, None, :], (TG, k, C1)).reshape(TG * k, C1)
    h1 = jnp.maximum((y1 - nwr) * sc1_ref[0:1, :] + sc1_ref[1:2, :], 0.0)
    y2 = jnp.dot(h1, w2t_ref[...], preferred_element_type=jnp.float32)
    C2 = y2.shape[1]
    o = jnp.maximum(y2 * sc2_ref[0:1, :] + sc2_ref[1:2, :], 0.0)
    out_ref[...] = jnp.max(o.reshape(TG, k, C2), axis=1)


def _run_mlp(gr, nxyz8, w1pt, w1at, w2t, g1, b1, g2, b2, k, tr):
    P, D = gr.shape
    C1 = w1pt.shape[1]
    C2 = w2t.shape[1]
    TG = tr // k
    n_t = P // tr
    grid = (n_t,)
    gr_spec = pl.BlockSpec((tr, D), lambda t: (t, 0))
    nx_spec = pl.BlockSpec((TG, 8), lambda t: (t, 0))
    w1pt_spec = pl.BlockSpec(w1pt.shape, lambda t: (0, 0))
    w1at_spec = pl.BlockSpec(w1at.shape, lambda t: (0, 0))
    w2t_spec = pl.BlockSpec(w2t.shape, lambda t: (0, 0))
    st_spec = pl.BlockSpec((8, C1), lambda t: (0, 0))

    st1 = pl.pallas_call(
        functools.partial(_mlp_stats1_body, k=k),
        grid=grid,
        in_specs=[gr_spec, nx_spec, w1pt_spec, w1at_spec],
        out_specs=st_spec,
        out_shape=jax.ShapeDtypeStruct((8, C1), jnp.float32),
    )(gr, nxyz8, w1pt, w1at)

    n = jnp.float32(P)
    mean1 = st1[0] / n
    var1 = st1[1] / n - mean1 * mean1
    scale1 = g1 / jnp.sqrt(var1 + 1e-5)
    shift1 = b1 - mean1 * scale1
    sc1 = jnp.stack([scale1, shift1], axis=0)

    st2_spec = pl.BlockSpec((8, C2), lambda t: (0, 0))
    sc1_spec = pl.BlockSpec((2, C1), lambda t: (0, 0))
    st2 = pl.pallas_call(
        functools.partial(_mlp_stats2_body, k=k),
        grid=grid,
        in_specs=[gr_spec, nx_spec, w1pt_spec, w1at_spec, sc1_spec, w2t_spec],
        out_specs=st2_spec,
        out_shape=jax.ShapeDtypeStruct((8, C2), jnp.float32),
    )(gr, nxyz8, w1pt, w1at, sc1, w2t)

    mean2 = st2[0] / n
    var2 = st2[1] / n - mean2 * mean2
    scale2 = g2 / jnp.sqrt(var2 + 1e-5)
    shift2 = b2 - mean2 * scale2
    sc2 = jnp.stack([scale2, shift2], axis=0)

    sc2_spec = pl.BlockSpec((2, C2), lambda t: (0, 0))
    out = pl.pallas_call(
        functools.partial(_mlp_final_body, k=k),
        grid=grid,
        in_specs=[gr_spec, nx_spec, w1pt_spec, w1at_spec, sc1_spec, w2t_spec,
                  sc2_spec],
        out_specs=pl.BlockSpec((TG, C2), lambda t: (t, 0)),
        out_shape=jax.ShapeDtypeStruct((P // k, C2), jnp.float32),
    )(gr, nxyz8, w1pt, w1at, sc1, w2t, sc2)
    return out


def kernel(xyz, features, W1, g1, b1, W2, g2, b2):
    B, N, _ = xyz.shape
    C = features.shape[1]
    S = NPOINT
    cent, nx, ny, nz = _run_fps(xyz, S)
    gidx = _run_knn(nx, ny, nz, xyz, K, min(64, S))

    Dpad = 128
    feat_t = jnp.transpose(features, (0, 2, 1))
    tbl = jnp.concatenate(
        [feat_t, xyz, jnp.zeros((B, N, Dpad - C - 3), jnp.float32)],
        axis=-1).reshape(B * N, Dpad)
    flat_idx = (gidx + (jnp.arange(B, dtype=jnp.int32) * N)[:, None, None]
                ).reshape(B * S * K)
    gr = _run_sc_gather(tbl, flat_idx)

    C1 = W1.shape[0]
    C2 = W2.shape[0]
    w1pt = jnp.zeros((Dpad, C1), jnp.float32)
    w1pt = w1pt.at[:C].set(W1[:, 3:].T)
    w1pt = w1pt.at[C:C + 3].set(W1[:, :3].T)
    w1at = jnp.zeros((8, C1), jnp.float32).at[:3].set(W1[:, :3].T)
    w2t = W2.T
    nxyz8 = jnp.zeros((B * S, 8), jnp.float32)
    nxyz8 = nxyz8.at[:, 0].set(nx.reshape(-1))
    nxyz8 = nxyz8.at[:, 1].set(ny.reshape(-1))
    nxyz8 = nxyz8.at[:, 2].set(nz.reshape(-1))

    pooled = _run_mlp(gr, nxyz8, w1pt, w1at, w2t, g1, b1, g2, b2, K,
                      min(4096, B * S * K))

    new_xyz = jnp.stack([nx, ny, nz], axis=-1)
    new_features = pooled.reshape(B, S, C2).transpose(0, 2, 1)
    return (new_xyz, new_features)


# speedup vs baseline: 11.0300x; 1.0068x over previous
import functools

import jax
import jax.numpy as jnp
from jax import lax
from jax.experimental import pallas as pl
from jax.experimental.pallas import tpu as pltpu
from jax.experimental.pallas import tpu_sc as plsc

NPOINT = 2048
K = 32


def _fps_body(x_ref, y_ref, z_ref, far0_ref, cent_ref, nx_ref, ny_ref, nz_ref,
              dist_ref, *, npoint):
    B, N = x_ref.shape
    col = lax.broadcasted_iota(jnp.int32, (B, N), 1)
    dist_ref[...] = jnp.full((B, N), 1e10, jnp.float32)
    x = x_ref[...]
    y = y_ref[...]
    z = z_ref[...]

    blk = 128
    lane = lax.broadcasted_iota(jnp.int32, (B, blk), 1)

    def step(i, carry):
        far, hc, hx, hy, hz = carry
        sel = col == far
        cx = jnp.min(jnp.where(sel, x, jnp.float32(1e30)), axis=1, keepdims=True)
        cy = jnp.min(jnp.where(sel, y, jnp.float32(1e30)), axis=1, keepdims=True)
        cz = jnp.min(jnp.where(sel, z, jnp.float32(1e30)), axis=1, keepdims=True)
        m = lane == i
        mf = m.astype(jnp.float32)
        hc = hc + far * m.astype(jnp.int32)
        hx = hx + cx * mf
        hy = hy + cy * mf
        hz = hz + cz * mf
        dx = x - cx
        dy = y - cy
        dz = z - cz
        d = dx * dx + dy * dy + dz * dz
        nd = jnp.minimum(dist_ref[...], d)
        dist_ref[...] = nd
        mx = jnp.max(nd, axis=1, keepdims=True)
        nf = jnp.min(jnp.where(nd == mx, col, jnp.int32(N)), axis=1,
                     keepdims=True)
        return nf, hc, hx, hy, hz

    far = far0_ref[...]
    for b in range(npoint // blk):
        hc = jnp.zeros((B, blk), jnp.int32)
        hf = jnp.zeros((B, blk), jnp.float32)
        far, hc, hx, hy, hz = lax.fori_loop(0, blk, step,
                                            (far, hc, hf, hf, hf))
        cent_ref[:, b * blk:(b + 1) * blk] = hc
        nx_ref[:, b * blk:(b + 1) * blk] = hx
        ny_ref[:, b * blk:(b + 1) * blk] = hy
        nz_ref[:, b * blk:(b + 1) * blk] = hz


def _run_fps(xyz, npoint):
    B, N, _ = xyz.shape
    xt = jnp.transpose(xyz, (0, 2, 1))
    far0 = jax.random.randint(jax.random.key(1), (B,), 0, N)
    far0 = far0.astype(jnp.int32).reshape(B, 1)
    out_shapes = (
        jax.ShapeDtypeStruct((B, npoint), jnp.int32),
        jax.ShapeDtypeStruct((B, npoint), jnp.float32),
        jax.ShapeDtypeStruct((B, npoint), jnp.float32),
        jax.ShapeDtypeStruct((B, npoint), jnp.float32),
    )
    return pl.pallas_call(
        functools.partial(_fps_body, npoint=npoint),
        out_shape=out_shapes,
        scratch_shapes=[pltpu.VMEM((B, N), jnp.float32)],
    )(xt[:, 0], xt[:, 1], xt[:, 2], far0)


def _knn_body(nx_ref, ny_ref, nz_ref, x_ref, idx_ref, dist_ref, *, k):
    TS = nx_ref.shape[1]
    N = x_ref.shape[2]
    cx = nx_ref[0]
    cy = ny_ref[0]
    cz = nz_ref[0]
    pts = x_ref[0]
    px = pts[0:1]
    py = pts[1:2]
    pz = pts[2:3]
    s2 = cx * cx + cy * cy + cz * cz
    d2 = px * px + py * py + pz * pz
    q = jnp.concatenate([cx, cy, cz], axis=1)
    cross = jax.lax.dot_general(
        q.astype(jnp.bfloat16), pts.astype(jnp.bfloat16),
        (((1,), (0,)), ((), ())), preferred_element_type=jnp.float32)
    dist_ref[...] = jnp.maximum(s2 + d2 - 2.0 * cross, 0.0)
    col = lax.broadcasted_iota(jnp.int32, (TS, N), 1)
    kcol = lax.broadcasted_iota(jnp.int32, (TS, k), 1)

    def step(j, carry):
        a_prev, acc = carry
        d = jnp.where(col == a_prev, jnp.float32(1e30), dist_ref[...])
        dist_ref[...] = d
        m = jnp.min(d, axis=1, keepdims=True)
        a = jnp.min(jnp.where(d == m, col, jnp.int32(N)), axis=1,
                    keepdims=True)
        return a, acc + a * (kcol == j).astype(jnp.int32)

    a0 = jnp.full((TS, 1), N, jnp.int32)
    _, acc = lax.fori_loop(0, k, step, (a0, jnp.zeros((TS, k), jnp.int32)))
    idx_ref[0] = acc


def _run_knn(nx, ny, nz, xyz, k, ts):
    B, S = nx.shape
    N = xyz.shape[1]
    xt = jnp.transpose(xyz, (0, 2, 1))
    nx3 = nx.reshape(B, S, 1)
    ny3 = ny.reshape(B, S, 1)
    nz3 = nz.reshape(B, S, 1)
    grid = (B, S // ts)
    q_spec = pl.BlockSpec((1, ts, 1), lambda b, s: (b, s, 0))
    return pl.pallas_call(
        functools.partial(_knn_body, k=k),
        grid=grid,
        in_specs=[q_spec, q_spec, q_spec,
                  pl.BlockSpec((1, 3, N), lambda b, s: (b, 0, 0))],
        out_specs=pl.BlockSpec((1, ts, k), lambda b, s: (b, s, 0)),
        out_shape=jax.ShapeDtypeStruct((B, S, k), jnp.int32),
        scratch_shapes=[pltpu.VMEM((ts, N), jnp.float32)],
    )(nx3, ny3, nz3, xt)


def _run_sc_gather(tbl, gidx):
    P = gidx.shape[0]
    D = tbl.shape[1]
    info = plsc.get_sparse_core_info()
    NC, NS = info.num_cores, info.num_subcores
    NW = NC * NS
    per_w = P // NW
    CH = 128
    n_chunks = per_w // CH
    mesh = plsc.VectorSubcoreMesh(core_axis_name="c", subcore_axis_name="s")

    @functools.partial(
        pl.kernel, mesh=mesh,
        out_type=jax.ShapeDtypeStruct((P, D), jnp.float32),
        scratch_types=[
            pltpu.VMEM((CH,), jnp.int32),
            pltpu.VMEM((CH, D), jnp.float32),
            pltpu.SemaphoreType.DMA,
        ],
    )
    def gather_k(tbl_hbm, idx_hbm, out_hbm, idx_v, rows_v, sem):
        wid = lax.axis_index("s") * NC + lax.axis_index("c")

        def chunk(c, _):
            base = wid * per_w + c * CH
            pltpu.sync_copy(idx_hbm.at[pl.ds(base, CH)], idx_v)
            pltpu.async_copy(tbl_hbm.at[idx_v], rows_v, sem).wait()
            pltpu.sync_copy(rows_v, out_hbm.at[pl.ds(base, CH)])
            return 0

        lax.fori_loop(0, n_chunks, chunk, 0)

    return gather_k(tbl, gidx)


def _mlp_stats1_body(gr_ref, nx_ref, w1pt_ref, w1at_ref, st_ref, *, k):
    t = pl.program_id(0)
    g = gr_ref[...]
    TG = nx_ref.shape[0]
    y1 = jnp.dot(g, w1pt_ref[...], preferred_element_type=jnp.float32)
    nw = jnp.dot(nx_ref[...], w1at_ref[...],
                 preferred_element_type=jnp.float32)
    C1 = y1.shape[1]
    nwr = jnp.broadcast_to(nw[:, None, :], (TG, k, C1)).reshape(TG * k, C1)
    y1 = y1 - nwr
    s = jnp.sum(y1, axis=0, keepdims=True)
    q = jnp.sum(y1 * y1, axis=0, keepdims=True)
    contrib = jnp.concatenate(
        [s, q, jnp.zeros((6, C1), jnp.float32)], axis=0)

    @pl.when(t == 0)
    def _():
        st_ref[...] = jnp.zeros_like(st_ref)

    st_ref[...] += contrib


def _mlp_stats2_body(gr_ref, nx_ref, w1pt_ref, w1at_ref, sc1_ref, w2t_ref,
                     st_ref, *, k):
    t = pl.program_id(0)
    g = gr_ref[...]
    TG = nx_ref.shape[0]
    y1 = jnp.dot(g, w1pt_ref[...], preferred_element_type=jnp.float32)
    nw = jnp.dot(nx_ref[...], w1at_ref[...],
                 preferred_element_type=jnp.float32)
    C1 = y1.shape[1]
    nwr = jnp.broadcast_to(nw[:, None, :], (TG, k, C1)).reshape(TG * k, C1)
    h1 = jnp.maximum((y1 - nwr) * sc1_ref[0:1, :] + sc1_ref[1:2, :], 0.0)
    y2 = jnp.dot(h1, w2t_ref[...], preferred_element_type=jnp.float32)
    C2 = y2.shape[1]
    s = jnp.sum(y2, axis=0, keepdims=True)
    q = jnp.sum(y2 * y2, axis=0, keepdims=True)
    contrib = jnp.concatenate(
        [s, q, jnp.zeros((6, C2), jnp.float32)], axis=0)

    @pl.when(t == 0)
    def _():
        st_ref[...] = jnp.zeros_like(st_ref)

    st_ref[...] += contrib


def _mlp_final_body(gr_ref, nx_ref, w1pt_ref, w1at_ref, sc1_ref, w2t_ref,
                    sc2_ref, out_ref, *, k):
    g = gr_ref[...]
    TG = nx_ref.shape[0]
    y1 = jnp.dot(g, w1pt_ref[...], preferred_element_type=jnp.float32)
    nw = jnp.dot(nx_ref[...], w1at_ref[...],
                 preferred_element_type=jnp.float32)
    C1 = y1.shape[1]
    nwr = jnp.broadcast_to(nw[:, None, :], (TG, k, C1)).reshape(TG * k, C1)
    h1 = jnp.maximum((y1 - nwr) * sc1_ref[0:1, :] + sc1_ref[1:2, :], 0.0)
    y2 = jnp.dot(h1, w2t_ref[...], preferred_element_type=jnp.float32)
    C2 = y2.shape[1]
    o = jnp.maximum(y2 * sc2_ref[0:1, :] + sc2_ref[1:2, :], 0.0)
    out_ref[...] = jnp.max(o.reshape(TG, k, C2), axis=1)


def _run_mlp(gr, nxyz8, w1pt, w1at, w2t, g1, b1, g2, b2, k, tr):
    P, D = gr.shape
    C1 = w1pt.shape[1]
    C2 = w2t.shape[1]
    TG = tr // k
    n_t = P // tr
    grid = (n_t,)
    gr_spec = pl.BlockSpec((tr, D), lambda t: (t, 0))
    nx_spec = pl.BlockSpec((TG, 8), lambda t: (t, 0))
    w1pt_spec = pl.BlockSpec(w1pt.shape, lambda t: (0, 0))
    w1at_spec = pl.BlockSpec(w1at.shape, lambda t: (0, 0))
    w2t_spec = pl.BlockSpec(w2t.shape, lambda t: (0, 0))
    st_spec = pl.BlockSpec((8, C1), lambda t: (0, 0))

    st1 = pl.pallas_call(
        functools.partial(_mlp_stats1_body, k=k),
        grid=grid,
        in_specs=[gr_spec, nx_spec, w1pt_spec, w1at_spec],
        out_specs=st_spec,
        out_shape=jax.ShapeDtypeStruct((8, C1), jnp.float32),
    )(gr, nxyz8, w1pt, w1at)

    n = jnp.float32(P)
    mean1 = st1[0] / n
    var1 = st1[1] / n - mean1 * mean1
    scale1 = g1 / jnp.sqrt(var1 + 1e-5)
    shift1 = b1 - mean1 * scale1
    sc1 = jnp.stack([scale1, shift1], axis=0)

    st2_spec = pl.BlockSpec((8, C2), lambda t: (0, 0))
    sc1_spec = pl.BlockSpec((2, C1), lambda t: (0, 0))
    st2 = pl.pallas_call(
        functools.partial(_mlp_stats2_body, k=k),
        grid=grid,
        in_specs=[gr_spec, nx_spec, w1pt_spec, w1at_spec, sc1_spec, w2t_spec],
        out_specs=st2_spec,
        out_shape=jax.ShapeDtypeStruct((8, C2), jnp.float32),
    )(gr, nxyz8, w1pt, w1at, sc1, w2t)

    mean2 = st2[0] / n
    var2 = st2[1] / n - mean2 * mean2
    scale2 = g2 / jnp.sqrt(var2 + 1e-5)
    shift2 = b2 - mean2 * scale2
    sc2 = jnp.stack([scale2, shift2], axis=0)

    sc2_spec = pl.BlockSpec((2, C2), lambda t: (0, 0))
    out = pl.pallas_call(
        functools.partial(_mlp_final_body, k=k),
        grid=grid,
        in_specs=[gr_spec, nx_spec, w1pt_spec, w1at_spec, sc1_spec, w2t_spec,
                  sc2_spec],
        out_specs=pl.BlockSpec((TG, C2), lambda t: (t, 0)),
        out_shape=jax.ShapeDtypeStruct((P // k, C2), jnp.float32),
    )(gr, nxyz8, w1pt, w1at, sc1, w2t, sc2)
    return out


def kernel(xyz, features, W1, g1, b1, W2, g2, b2):
    B, N, _ = xyz.shape
    C = features.shape[1]
    S = NPOINT
    cent, nx, ny, nz = _run_fps(xyz, S)
    gidx = _run_knn(nx, ny, nz, xyz, K, min(64, S))

    Dpad = 128
    feat_t = jnp.transpose(features, (0, 2, 1))
    tbl = jnp.concatenate(
        [feat_t, xyz, jnp.zeros((B, N, Dpad - C - 3), jnp.float32)],
        axis=-1).reshape(B * N, Dpad)
    flat_idx = (gidx + (jnp.arange(B, dtype=jnp.int32) * N)[:, None, None]
                ).reshape(B * S * K)
    gr = _run_sc_gather(tbl, flat_idx)

    C1 = W1.shape[0]
    C2 = W2.shape[0]
    w1pt = jnp.zeros((Dpad, C1), jnp.float32)
    w1pt = w1pt.at[:C].set(W1[:, 3:].T)
    w1pt = w1pt.at[C:C + 3].set(W1[:, :3].T)
    w1at = jnp.zeros((8, C1), jnp.float32).at[:3].set(W1[:, :3].T)
    w2t = W2.T
    nxyz8 = jnp.zeros((B * S, 8), jnp.float32)
    nxyz8 = nxyz8.at[:, 0].set(nx.reshape(-1))
    nxyz8 = nxyz8.at[:, 1].set(ny.reshape(-1))
    nxyz8 = nxyz8.at[:, 2].set(nz.reshape(-1))

    pooled = _run_mlp(gr, nxyz8, w1pt, w1at, w2t, g1, b1, g2, b2, K,
                      min(4096, B * S * K))

    new_xyz = jnp.stack([nx, ny, nz], axis=-1)
    new_features = pooled.reshape(B, S, C2).transpose(0, 2, 1)
    return (new_xyz, new_features)


# speedup vs baseline: 11.8820x; 1.0772x over previous
import functools

import jax
import jax.numpy as jnp
from jax import lax
from jax.experimental import pallas as pl
from jax.experimental.pallas import tpu as pltpu
from jax.experimental.pallas import tpu_sc as plsc

NPOINT = 2048
K = 32


def _fps_body(x_ref, y_ref, z_ref, far0_ref, cent_ref, nx_ref, ny_ref, nz_ref,
              dist_ref, *, npoint):
    B, N = x_ref.shape
    col = lax.broadcasted_iota(jnp.int32, (B, N), 1)
    dist_ref[...] = jnp.full((B, N), 1e10, jnp.float32)
    x = x_ref[...]
    y = y_ref[...]
    z = z_ref[...]

    blk = 128
    lane = lax.broadcasted_iota(jnp.int32, (B, blk), 1)

    def step(i, carry):
        far, hc, hx, hy, hz = carry
        sel = col == far
        cx = jnp.min(jnp.where(sel, x, jnp.float32(1e30)), axis=1, keepdims=True)
        cy = jnp.min(jnp.where(sel, y, jnp.float32(1e30)), axis=1, keepdims=True)
        cz = jnp.min(jnp.where(sel, z, jnp.float32(1e30)), axis=1, keepdims=True)
        m = lane == i
        mf = m.astype(jnp.float32)
        hc = hc + far * m.astype(jnp.int32)
        hx = hx + cx * mf
        hy = hy + cy * mf
        hz = hz + cz * mf
        dx = x - cx
        dy = y - cy
        dz = z - cz
        d = dx * dx + dy * dy + dz * dz
        nd = jnp.minimum(dist_ref[...], d)
        dist_ref[...] = nd
        mx = jnp.max(nd, axis=1, keepdims=True)
        nf = jnp.min(jnp.where(nd == mx, col, jnp.int32(N)), axis=1,
                     keepdims=True)
        return nf, hc, hx, hy, hz

    far = far0_ref[...]
    for b in range(npoint // blk):
        hc = jnp.zeros((B, blk), jnp.int32)
        hf = jnp.zeros((B, blk), jnp.float32)
        far, hc, hx, hy, hz = lax.fori_loop(0, blk, step,
                                            (far, hc, hf, hf, hf))
        cent_ref[:, b * blk:(b + 1) * blk] = hc
        nx_ref[:, b * blk:(b + 1) * blk] = hx
        ny_ref[:, b * blk:(b + 1) * blk] = hy
        nz_ref[:, b * blk:(b + 1) * blk] = hz


def _run_fps(xyz, npoint):
    B, N, _ = xyz.shape
    xt = jnp.transpose(xyz, (0, 2, 1))
    far0 = jax.random.randint(jax.random.key(1), (B,), 0, N)
    far0 = far0.astype(jnp.int32).reshape(B, 1)
    out_shapes = (
        jax.ShapeDtypeStruct((B, npoint), jnp.int32),
        jax.ShapeDtypeStruct((B, npoint), jnp.float32),
        jax.ShapeDtypeStruct((B, npoint), jnp.float32),
        jax.ShapeDtypeStruct((B, npoint), jnp.float32),
    )
    return pl.pallas_call(
        functools.partial(_fps_body, npoint=npoint),
        out_shape=out_shapes,
        scratch_shapes=[pltpu.VMEM((B, N), jnp.float32)],
    )(xt[:, 0], xt[:, 1], xt[:, 2], far0)


def _knn_body(nx_ref, ny_ref, nz_ref, x_ref, idx_ref, dist_ref, *, k):
    TS = nx_ref.shape[1]
    N = x_ref.shape[2]
    cx = nx_ref[0]
    cy = ny_ref[0]
    cz = nz_ref[0]
    pts = x_ref[0]
    px = pts[0:1]
    py = pts[1:2]
    pz = pts[2:3]
    s2 = cx * cx + cy * cy + cz * cz
    d2 = px * px + py * py + pz * pz
    q = jnp.concatenate([cx, cy, cz], axis=1)
    cross = jax.lax.dot_general(
        q.astype(jnp.bfloat16), pts.astype(jnp.bfloat16),
        (((1,), (0,)), ((), ())), preferred_element_type=jnp.float32)
    dist_ref[...] = jnp.maximum(s2 + d2 - 2.0 * cross, 0.0)
    col = lax.broadcasted_iota(jnp.int32, (TS, N), 1)
    kcol = lax.broadcasted_iota(jnp.int32, (TS, k), 1)

    def step(j, carry):
        a_prev, acc = carry
        d = jnp.where(col == a_prev, jnp.float32(1e30), dist_ref[...])
        dist_ref[...] = d
        m = jnp.min(d, axis=1, keepdims=True)
        a = jnp.min(jnp.where(d == m, col, jnp.int32(N)), axis=1,
                    keepdims=True)
        return a, acc + a * (kcol == j).astype(jnp.int32)

    a0 = jnp.full((TS, 1), N, jnp.int32)
    _, acc = lax.fori_loop(0, k, step, (a0, jnp.zeros((TS, k), jnp.int32)))
    idx_ref[0] = acc


def _run_knn(nx, ny, nz, xyz, k, ts):
    B, S = nx.shape
    N = xyz.shape[1]
    xt = jnp.transpose(xyz, (0, 2, 1))
    nx3 = nx.reshape(B, S, 1)
    ny3 = ny.reshape(B, S, 1)
    nz3 = nz.reshape(B, S, 1)
    grid = (B, S // ts)
    q_spec = pl.BlockSpec((1, ts, 1), lambda b, s: (b, s, 0))
    return pl.pallas_call(
        functools.partial(_knn_body, k=k),
        grid=grid,
        in_specs=[q_spec, q_spec, q_spec,
                  pl.BlockSpec((1, 3, N), lambda b, s: (b, 0, 0))],
        out_specs=pl.BlockSpec((1, ts, k), lambda b, s: (b, s, 0)),
        out_shape=jax.ShapeDtypeStruct((B, S, k), jnp.int32),
        scratch_shapes=[pltpu.VMEM((ts, N), jnp.float32)],
    )(nx3, ny3, nz3, xt)


def _run_sc_gather(tbl, gidx):
    P = gidx.shape[0]
    D = tbl.shape[1]
    info = plsc.get_sparse_core_info()
    NC, NS = info.num_cores, info.num_subcores
    NW = NC * NS
    per_w = P // NW
    CH = 128
    n_chunks = per_w // CH
    mesh = plsc.VectorSubcoreMesh(core_axis_name="c", subcore_axis_name="s")

    @functools.partial(
        pl.kernel, mesh=mesh,
        out_type=jax.ShapeDtypeStruct((P, D), jnp.float32),
        scratch_types=[
            pltpu.VMEM((CH,), jnp.int32),
            pltpu.VMEM((CH, D), jnp.float32),
            pltpu.SemaphoreType.DMA,
        ],
    )
    def gather_k(tbl_hbm, idx_hbm, out_hbm, idx_v, rows_v, sem):
        wid = lax.axis_index("s") * NC + lax.axis_index("c")

        def chunk(c, _):
            base = wid * per_w + c * CH
            pltpu.sync_copy(idx_hbm.at[pl.ds(base, CH)], idx_v)
            pltpu.async_copy(tbl_hbm.at[idx_v], rows_v, sem).wait()
            pltpu.sync_copy(rows_v, out_hbm.at[pl.ds(base, CH)])
            return 0

        lax.fori_loop(0, n_chunks, chunk, 0)

    return gather_k(tbl, gidx)


def _mlp_stats1_body(gr_ref, nx_ref, w1pt_ref, w1at_ref, st_ref, *, k):
    t = pl.program_id(0)
    g = gr_ref[...]
    TG = nx_ref.shape[0]
    y1 = jnp.dot(g, w1pt_ref[...], preferred_element_type=jnp.float32)
    nw = jnp.dot(nx_ref[...], w1at_ref[...],
                 preferred_element_type=jnp.float32)
    C1 = y1.shape[1]
    nwr = jnp.broadcast_to(nw[:, None, :], (TG, k, C1)).reshape(TG * k, C1)
    y1 = y1 - nwr
    s = jnp.sum(y1, axis=0, keepdims=True)
    q = jnp.sum(y1 * y1, axis=0, keepdims=True)
    contrib = jnp.concatenate(
        [s, q, jnp.zeros((6, C1), jnp.float32)], axis=0)

    @pl.when(t == 0)
    def _():
        st_ref[...] = jnp.zeros_like(st_ref)

    st_ref[...] += contrib


def _mlp_stats2_body(gr_ref, nx_ref, w1pt_ref, w1at_ref, sc1_ref, w2t_ref,
                     st_ref, *, k):
    t = pl.program_id(0)
    g = gr_ref[...]
    TG = nx_ref.shape[0]
    y1 = jnp.dot(g, w1pt_ref[...], preferred_element_type=jnp.float32)
    nw = jnp.dot(nx_ref[...], w1at_ref[...],
                 preferred_element_type=jnp.float32)
    C1 = y1.shape[1]
    nwr = jnp.broadcast_to(nw[:, None, :], (TG, k, C1)).reshape(TG * k, C1)
    h1 = jnp.maximum((y1 - nwr) * sc1_ref[0:1, :] + sc1_ref[1:2, :], 0.0)
    y2 = jnp.dot(h1, w2t_ref[...], preferred_element_type=jnp.float32)
    C2 = y2.shape[1]
    s = jnp.sum(y2, axis=0, keepdims=True)
    q = jnp.sum(y2 * y2, axis=0, keepdims=True)
    contrib = jnp.concatenate(
        [s, q, jnp.zeros((6, C2), jnp.float32)], axis=0)

    @pl.when(t == 0)
    def _():
        st_ref[...] = jnp.zeros_like(st_ref)

    st_ref[...] += contrib


def _mlp_final_body(gr_ref, nx_ref, w1pt_ref, w1at_ref, sc1_ref, w2t_ref,
                    sc2_ref, out_ref, *, k):
    g = gr_ref[...]
    TG = nx_ref.shape[0]
    y1 = jnp.dot(g, w1pt_ref[...], preferred_element_type=jnp.float32)
    nw = jnp.dot(nx_ref[...], w1at_ref[...],
                 preferred_element_type=jnp.float32)
    C1 = y1.shape[1]
    nwr = jnp.broadcast_to(nw[:, None, :], (TG, k, C1)).reshape(TG * k, C1)
    h1 = jnp.maximum((y1 - nwr) * sc1_ref[0:1, :] + sc1_ref[1:2, :], 0.0)
    y2 = jnp.dot(h1, w2t_ref[...], preferred_element_type=jnp.float32)
    C2 = y2.shape[1]
    o = jnp.maximum(y2 * sc2_ref[0:1, :] + sc2_ref[1:2, :], 0.0)
    out_ref[...] = jnp.max(o.reshape(TG, k, C2), axis=1)


def _run_mlp(gr, nxyz8, w1pt, w1at, w2t, g1, b1, g2, b2, k, tr):
    P, D = gr.shape
    C1 = w1pt.shape[1]
    C2 = w2t.shape[1]
    TG = tr // k
    n_t = P // tr
    grid = (n_t,)
    gr_spec = pl.BlockSpec((tr, D), lambda t: (t, 0))
    nx_spec = pl.BlockSpec((TG, 8), lambda t: (t, 0))
    w1pt_spec = pl.BlockSpec(w1pt.shape, lambda t: (0, 0))
    w1at_spec = pl.BlockSpec(w1at.shape, lambda t: (0, 0))
    w2t_spec = pl.BlockSpec(w2t.shape, lambda t: (0, 0))
    st_spec = pl.BlockSpec((8, C1), lambda t: (0, 0))

    st1 = pl.pallas_call(
        functools.partial(_mlp_stats1_body, k=k),
        grid=grid,
        in_specs=[gr_spec, nx_spec, w1pt_spec, w1at_spec],
        out_specs=st_spec,
        out_shape=jax.ShapeDtypeStruct((8, C1), jnp.float32),
    )(gr, nxyz8, w1pt, w1at)

    n = jnp.float32(P)
    mean1 = st1[0] / n
    var1 = st1[1] / n - mean1 * mean1
    scale1 = g1 / jnp.sqrt(var1 + 1e-5)
    shift1 = b1 - mean1 * scale1
    sc1 = jnp.stack([scale1, shift1], axis=0)

    st2_spec = pl.BlockSpec((8, C2), lambda t: (0, 0))
    sc1_spec = pl.BlockSpec((2, C1), lambda t: (0, 0))
    st2 = pl.pallas_call(
        functools.partial(_mlp_stats2_body, k=k),
        grid=grid,
        in_specs=[gr_spec, nx_spec, w1pt_spec, w1at_spec, sc1_spec, w2t_spec],
        out_specs=st2_spec,
        out_shape=jax.ShapeDtypeStruct((8, C2), jnp.float32),
    )(gr, nxyz8, w1pt, w1at, sc1, w2t)

    mean2 = st2[0] / n
    var2 = st2[1] / n - mean2 * mean2
    scale2 = g2 / jnp.sqrt(var2 + 1e-5)
    shift2 = b2 - mean2 * scale2
    sc2 = jnp.stack([scale2, shift2], axis=0)

    sc2_spec = pl.BlockSpec((2, C2), lambda t: (0, 0))
    out = pl.pallas_call(
        functools.partial(_mlp_final_body, k=k),
        grid=grid,
        in_specs=[gr_spec, nx_spec, w1pt_spec, w1at_spec, sc1_spec, w2t_spec,
                  sc2_spec],
        out_specs=pl.BlockSpec((TG, C2), lambda t: (t, 0)),
        out_shape=jax.ShapeDtypeStruct((P // k, C2), jnp.float32),
    )(gr, nxyz8, w1pt, w1at, sc1, w2t, sc2)
    return out


def kernel(xyz, features, W1, g1, b1, W2, g2, b2):
    B, N, _ = xyz.shape
    C = features.shape[1]
    S = NPOINT
    cent, nx, ny, nz = _run_fps(xyz, S)
    gidx = _run_knn(nx, ny, nz, xyz, K, min(128, S))

    Dpad = 128
    feat_t = jnp.transpose(features, (0, 2, 1))
    tbl = jnp.concatenate(
        [feat_t, xyz, jnp.zeros((B, N, Dpad - C - 3), jnp.float32)],
        axis=-1).reshape(B * N, Dpad)
    flat_idx = (gidx + (jnp.arange(B, dtype=jnp.int32) * N)[:, None, None]
                ).reshape(B * S * K)
    gr = _run_sc_gather(tbl, flat_idx)

    C1 = W1.shape[0]
    C2 = W2.shape[0]
    w1pt = jnp.zeros((Dpad, C1), jnp.float32)
    w1pt = w1pt.at[:C].set(W1[:, 3:].T)
    w1pt = w1pt.at[C:C + 3].set(W1[:, :3].T)
    w1at = jnp.zeros((8, C1), jnp.float32).at[:3].set(W1[:, :3].T)
    w2t = W2.T
    nxyz8 = jnp.zeros((B * S, 8), jnp.float32)
    nxyz8 = nxyz8.at[:, 0].set(nx.reshape(-1))
    nxyz8 = nxyz8.at[:, 1].set(ny.reshape(-1))
    nxyz8 = nxyz8.at[:, 2].set(nz.reshape(-1))

    pooled = _run_mlp(gr, nxyz8, w1pt, w1at, w2t, g1, b1, g2, b2, K,
                      min(4096, B * S * K))

    new_xyz = jnp.stack([nx, ny, nz], axis=-1)
    new_features = pooled.reshape(B, S, C2).transpose(0, 2, 1)
    return (new_xyz, new_features)


# speedup vs baseline: 12.4990x; 1.0519x over previous
import functools

import jax
import jax.numpy as jnp
from jax import lax
from jax.experimental import pallas as pl
from jax.experimental.pallas import tpu as pltpu
from jax.experimental.pallas import tpu_sc as plsc

NPOINT = 2048
K = 32


def _fps_body(x_ref, y_ref, z_ref, far0_ref, cent_ref, nx_ref, ny_ref, nz_ref,
              dist_ref, *, npoint):
    B, N = x_ref.shape
    col = lax.broadcasted_iota(jnp.int32, (B, N), 1)
    dist_ref[...] = jnp.full((B, N), 1e10, jnp.float32)
    x = x_ref[...]
    y = y_ref[...]
    z = z_ref[...]

    blk = 128
    lane = lax.broadcasted_iota(jnp.int32, (B, blk), 1)

    def step(i, carry):
        far, hc, hx, hy, hz = carry
        sel = col == far
        cx = jnp.min(jnp.where(sel, x, jnp.float32(1e30)), axis=1, keepdims=True)
        cy = jnp.min(jnp.where(sel, y, jnp.float32(1e30)), axis=1, keepdims=True)
        cz = jnp.min(jnp.where(sel, z, jnp.float32(1e30)), axis=1, keepdims=True)
        m = lane == i
        mf = m.astype(jnp.float32)
        hc = hc + far * m.astype(jnp.int32)
        hx = hx + cx * mf
        hy = hy + cy * mf
        hz = hz + cz * mf
        dx = x - cx
        dy = y - cy
        dz = z - cz
        d = dx * dx + dy * dy + dz * dz
        nd = jnp.minimum(dist_ref[...], d)
        dist_ref[...] = nd
        mx = jnp.max(nd, axis=1, keepdims=True)
        nf = jnp.min(jnp.where(nd == mx, col, jnp.int32(N)), axis=1,
                     keepdims=True)
        return nf, hc, hx, hy, hz

    far = far0_ref[...]
    for b in range(npoint // blk):
        hc = jnp.zeros((B, blk), jnp.int32)
        hf = jnp.zeros((B, blk), jnp.float32)
        far, hc, hx, hy, hz = lax.fori_loop(0, blk, step,
                                            (far, hc, hf, hf, hf))
        cent_ref[:, b * blk:(b + 1) * blk] = hc
        nx_ref[:, b * blk:(b + 1) * blk] = hx
        ny_ref[:, b * blk:(b + 1) * blk] = hy
        nz_ref[:, b * blk:(b + 1) * blk] = hz


def _run_fps(xyz, npoint):
    B, N, _ = xyz.shape
    xt = jnp.transpose(xyz, (0, 2, 1))
    far0 = jax.random.randint(jax.random.key(1), (B,), 0, N)
    far0 = far0.astype(jnp.int32).reshape(B, 1)
    out_shapes = (
        jax.ShapeDtypeStruct((B, npoint), jnp.int32),
        jax.ShapeDtypeStruct((B, npoint), jnp.float32),
        jax.ShapeDtypeStruct((B, npoint), jnp.float32),
        jax.ShapeDtypeStruct((B, npoint), jnp.float32),
    )
    return pl.pallas_call(
        functools.partial(_fps_body, npoint=npoint),
        out_shape=out_shapes,
        scratch_shapes=[pltpu.VMEM((B, N), jnp.float32)],
    )(xt[:, 0], xt[:, 1], xt[:, 2], far0)


def _knn_body(nx_ref, ny_ref, nz_ref, x_ref, idx_ref, dist_ref, *, k):
    TS = nx_ref.shape[1]
    N = x_ref.shape[2]
    cx = nx_ref[0]
    cy = ny_ref[0]
    cz = nz_ref[0]
    pts = x_ref[0]
    px = pts[0:1]
    py = pts[1:2]
    pz = pts[2:3]
    s2 = cx * cx + cy * cy + cz * cz
    d2 = px * px + py * py + pz * pz
    q = jnp.concatenate([cx, cy, cz], axis=1)
    cross = jax.lax.dot_general(
        q.astype(jnp.bfloat16), pts.astype(jnp.bfloat16),
        (((1,), (0,)), ((), ())), preferred_element_type=jnp.float32)
    dist_ref[...] = jnp.maximum(s2 + d2 - 2.0 * cross, 0.0)
    col = lax.broadcasted_iota(jnp.int32, (TS, N), 1)
    kcol = lax.broadcasted_iota(jnp.int32, (TS, k), 1)

    def step(j, carry):
        a_prev, acc = carry
        d = jnp.where(col == a_prev, jnp.float32(1e30), dist_ref[...])
        dist_ref[...] = d
        m = jnp.min(d, axis=1, keepdims=True)
        a = jnp.min(jnp.where(d == m, col, jnp.int32(N)), axis=1,
                    keepdims=True)
        return a, acc + a * (kcol == j).astype(jnp.int32)

    a0 = jnp.full((TS, 1), N, jnp.int32)
    _, acc = lax.fori_loop(0, k, step, (a0, jnp.zeros((TS, k), jnp.int32)))
    idx_ref[0] = acc


def _run_knn(nx, ny, nz, xyz, k, ts):
    B, S = nx.shape
    N = xyz.shape[1]
    xt = jnp.transpose(xyz, (0, 2, 1))
    nx3 = nx.reshape(B, S, 1)
    ny3 = ny.reshape(B, S, 1)
    nz3 = nz.reshape(B, S, 1)
    grid = (B, S // ts)
    q_spec = pl.BlockSpec((1, ts, 1), lambda b, s: (b, s, 0))
    return pl.pallas_call(
        functools.partial(_knn_body, k=k),
        grid=grid,
        in_specs=[q_spec, q_spec, q_spec,
                  pl.BlockSpec((1, 3, N), lambda b, s: (b, 0, 0))],
        out_specs=pl.BlockSpec((1, ts, k), lambda b, s: (b, s, 0)),
        out_shape=jax.ShapeDtypeStruct((B, S, k), jnp.int32),
        scratch_shapes=[pltpu.VMEM((ts, N), jnp.float32)],
    )(nx3, ny3, nz3, xt)


def _run_sc_gather(tbl, gidx):
    P = gidx.shape[0]
    D = tbl.shape[1]
    info = plsc.get_sparse_core_info()
    NC, NS = info.num_cores, info.num_subcores
    NW = NC * NS
    per_w = P // NW
    CH = 128
    n_chunks = per_w // CH
    mesh = plsc.VectorSubcoreMesh(core_axis_name="c", subcore_axis_name="s")

    @functools.partial(
        pl.kernel, mesh=mesh,
        out_type=jax.ShapeDtypeStruct((P, D), jnp.float32),
        scratch_types=[
            pltpu.VMEM((CH,), jnp.int32),
            pltpu.VMEM((CH, D), jnp.float32),
            pltpu.SemaphoreType.DMA,
        ],
    )
    def gather_k(tbl_hbm, idx_hbm, out_hbm, idx_v, rows_v, sem):
        wid = lax.axis_index("s") * NC + lax.axis_index("c")

        def chunk(c, _):
            base = wid * per_w + c * CH
            pltpu.sync_copy(idx_hbm.at[pl.ds(base, CH)], idx_v)
            pltpu.async_copy(tbl_hbm.at[idx_v], rows_v, sem).wait()
            pltpu.sync_copy(rows_v, out_hbm.at[pl.ds(base, CH)])
            return 0

        lax.fori_loop(0, n_chunks, chunk, 0)

    return gather_k(tbl, gidx)


def _mlp_stats1_body(gr_ref, nx_ref, w1pt_ref, w1at_ref, st_ref, *, k):
    t = pl.program_id(0)
    g = gr_ref[...]
    TG = nx_ref.shape[0]
    y1 = jnp.dot(g, w1pt_ref[...], preferred_element_type=jnp.float32)
    nw = jnp.dot(nx_ref[...], w1at_ref[...],
                 preferred_element_type=jnp.float32)
    C1 = y1.shape[1]
    nwr = jnp.broadcast_to(nw[:, None, :], (TG, k, C1)).reshape(TG * k, C1)
    y1 = y1 - nwr
    s = jnp.sum(y1, axis=0, keepdims=True)
    q = jnp.sum(y1 * y1, axis=0, keepdims=True)
    contrib = jnp.concatenate(
        [s, q, jnp.zeros((6, C1), jnp.float32)], axis=0)

    @pl.when(t == 0)
    def _():
        st_ref[...] = jnp.zeros_like(st_ref)

    st_ref[...] += contrib


def _mlp_stats2_body(gr_ref, nx_ref, w1pt_ref, w1at_ref, sc1_ref, w2t_ref,
                     st_ref, *, k):
    t = pl.program_id(0)
    g = gr_ref[...]
    TG = nx_ref.shape[0]
    y1 = jnp.dot(g, w1pt_ref[...], preferred_element_type=jnp.float32)
    nw = jnp.dot(nx_ref[...], w1at_ref[...],
                 preferred_element_type=jnp.float32)
    C1 = y1.shape[1]
    nwr = jnp.broadcast_to(nw[:, None, :], (TG, k, C1)).reshape(TG * k, C1)
    h1 = jnp.maximum((y1 - nwr) * sc1_ref[0:1, :] + sc1_ref[1:2, :], 0.0)
    y2 = jnp.dot(h1, w2t_ref[...], preferred_element_type=jnp.float32)
    C2 = y2.shape[1]
    s = jnp.sum(y2, axis=0, keepdims=True)
    q = jnp.sum(y2 * y2, axis=0, keepdims=True)
    contrib = jnp.concatenate(
        [s, q, jnp.zeros((6, C2), jnp.float32)], axis=0)

    @pl.when(t == 0)
    def _():
        st_ref[...] = jnp.zeros_like(st_ref)

    st_ref[...] += contrib


def _mlp_final_body(gr_ref, nx_ref, w1pt_ref, w1at_ref, sc1_ref, w2t_ref,
                    sc2_ref, out_ref, *, k):
    g = gr_ref[...]
    TG = nx_ref.shape[0]
    y1 = jnp.dot(g, w1pt_ref[...], preferred_element_type=jnp.float32)
    nw = jnp.dot(nx_ref[...], w1at_ref[...],
                 preferred_element_type=jnp.float32)
    C1 = y1.shape[1]
    nwr = jnp.broadcast_to(nw[:, None, :], (TG, k, C1)).reshape(TG * k, C1)
    h1 = jnp.maximum((y1 - nwr) * sc1_ref[0:1, :] + sc1_ref[1:2, :], 0.0)
    y2 = jnp.dot(h1, w2t_ref[...], preferred_element_type=jnp.float32)
    C2 = y2.shape[1]
    o = jnp.maximum(y2 * sc2_ref[0:1, :] + sc2_ref[1:2, :], 0.0)
    out_ref[...] = jnp.max(o.reshape(TG, k, C2), axis=1)


def _run_mlp(gr, nxyz8, w1pt, w1at, w2t, g1, b1, g2, b2, k, tr):
    P, D = gr.shape
    C1 = w1pt.shape[1]
    C2 = w2t.shape[1]
    TG = tr // k
    n_t = P // tr
    grid = (n_t,)
    gr_spec = pl.BlockSpec((tr, D), lambda t: (t, 0))
    nx_spec = pl.BlockSpec((TG, 8), lambda t: (t, 0))
    w1pt_spec = pl.BlockSpec(w1pt.shape, lambda t: (0, 0))
    w1at_spec = pl.BlockSpec(w1at.shape, lambda t: (0, 0))
    w2t_spec = pl.BlockSpec(w2t.shape, lambda t: (0, 0))
    st_spec = pl.BlockSpec((8, C1), lambda t: (0, 0))

    st1 = pl.pallas_call(
        functools.partial(_mlp_stats1_body, k=k),
        grid=grid,
        in_specs=[gr_spec, nx_spec, w1pt_spec, w1at_spec],
        out_specs=st_spec,
        out_shape=jax.ShapeDtypeStruct((8, C1), jnp.float32),
    )(gr, nxyz8, w1pt, w1at)

    n = jnp.float32(P)
    mean1 = st1[0] / n
    var1 = st1[1] / n - mean1 * mean1
    scale1 = g1 / jnp.sqrt(var1 + 1e-5)
    shift1 = b1 - mean1 * scale1
    sc1 = jnp.stack([scale1, shift1], axis=0)

    st2_spec = pl.BlockSpec((8, C2), lambda t: (0, 0))
    sc1_spec = pl.BlockSpec((2, C1), lambda t: (0, 0))
    st2 = pl.pallas_call(
        functools.partial(_mlp_stats2_body, k=k),
        grid=grid,
        in_specs=[gr_spec, nx_spec, w1pt_spec, w1at_spec, sc1_spec, w2t_spec],
        out_specs=st2_spec,
        out_shape=jax.ShapeDtypeStruct((8, C2), jnp.float32),
    )(gr, nxyz8, w1pt, w1at, sc1, w2t)

    mean2 = st2[0] / n
    var2 = st2[1] / n - mean2 * mean2
    scale2 = g2 / jnp.sqrt(var2 + 1e-5)
    shift2 = b2 - mean2 * scale2
    sc2 = jnp.stack([scale2, shift2], axis=0)

    sc2_spec = pl.BlockSpec((2, C2), lambda t: (0, 0))
    out = pl.pallas_call(
        functools.partial(_mlp_final_body, k=k),
        grid=grid,
        in_specs=[gr_spec, nx_spec, w1pt_spec, w1at_spec, sc1_spec, w2t_spec,
                  sc2_spec],
        out_specs=pl.BlockSpec((TG, C2), lambda t: (t, 0)),
        out_shape=jax.ShapeDtypeStruct((P // k, C2), jnp.float32),
    )(gr, nxyz8, w1pt, w1at, sc1, w2t, sc2)
    return out


def kernel(xyz, features, W1, g1, b1, W2, g2, b2):
    B, N, _ = xyz.shape
    C = features.shape[1]
    S = NPOINT
    cent, nx, ny, nz = _run_fps(xyz, S)
    gidx = _run_knn(nx, ny, nz, xyz, K, min(256, S))

    Dpad = 128
    feat_t = jnp.transpose(features, (0, 2, 1))
    tbl = jnp.concatenate(
        [feat_t, xyz, jnp.zeros((B, N, Dpad - C - 3), jnp.float32)],
        axis=-1).reshape(B * N, Dpad)
    flat_idx = (gidx + (jnp.arange(B, dtype=jnp.int32) * N)[:, None, None]
                ).reshape(B * S * K)
    gr = _run_sc_gather(tbl, flat_idx)

    C1 = W1.shape[0]
    C2 = W2.shape[0]
    w1pt = jnp.zeros((Dpad, C1), jnp.float32)
    w1pt = w1pt.at[:C].set(W1[:, 3:].T)
    w1pt = w1pt.at[C:C + 3].set(W1[:, :3].T)
    w1at = jnp.zeros((8, C1), jnp.float32).at[:3].set(W1[:, :3].T)
    w2t = W2.T
    nxyz8 = jnp.zeros((B * S, 8), jnp.float32)
    nxyz8 = nxyz8.at[:, 0].set(nx.reshape(-1))
    nxyz8 = nxyz8.at[:, 1].set(ny.reshape(-1))
    nxyz8 = nxyz8.at[:, 2].set(nz.reshape(-1))

    pooled = _run_mlp(gr, nxyz8, w1pt, w1at, w2t, g1, b1, g2, b2, K,
                      min(4096, B * S * K))

    new_xyz = jnp.stack([nx, ny, nz], axis=-1)
    new_features = pooled.reshape(B, S, C2).transpose(0, 2, 1)
    return (new_xyz, new_features)


# speedup vs baseline: 12.7648x; 1.0213x over previous
import functools

import jax
import jax.numpy as jnp
from jax import lax
from jax.experimental import pallas as pl
from jax.experimental.pallas import tpu as pltpu
from jax.experimental.pallas import tpu_sc as plsc

NPOINT = 2048
K = 32


def _fps_body(x_ref, y_ref, z_ref, far0_ref, cent_ref, nx_ref, ny_ref, nz_ref,
              dist_ref, *, npoint):
    B, N = x_ref.shape
    col = lax.broadcasted_iota(jnp.int32, (B, N), 1)
    dist_ref[...] = jnp.full((B, N), 1e10, jnp.float32)
    x = x_ref[...]
    y = y_ref[...]
    z = z_ref[...]

    blk = 128
    lane = lax.broadcasted_iota(jnp.int32, (B, blk), 1)

    def step(i, carry):
        far, hc, hx, hy, hz = carry
        sel = col == far
        cx = jnp.min(jnp.where(sel, x, jnp.float32(1e30)), axis=1, keepdims=True)
        cy = jnp.min(jnp.where(sel, y, jnp.float32(1e30)), axis=1, keepdims=True)
        cz = jnp.min(jnp.where(sel, z, jnp.float32(1e30)), axis=1, keepdims=True)
        m = lane == i
        mf = m.astype(jnp.float32)
        hc = hc + far * m.astype(jnp.int32)
        hx = hx + cx * mf
        hy = hy + cy * mf
        hz = hz + cz * mf
        dx = x - cx
        dy = y - cy
        dz = z - cz
        d = dx * dx + dy * dy + dz * dz
        nd = jnp.minimum(dist_ref[...], d)
        dist_ref[...] = nd
        mx = jnp.max(nd, axis=1, keepdims=True)
        nf = jnp.min(jnp.where(nd == mx, col, jnp.int32(N)), axis=1,
                     keepdims=True)
        return nf, hc, hx, hy, hz

    far = far0_ref[...]
    for b in range(npoint // blk):
        hc = jnp.zeros((B, blk), jnp.int32)
        hf = jnp.zeros((B, blk), jnp.float32)
        far, hc, hx, hy, hz = lax.fori_loop(0, blk, step,
                                            (far, hc, hf, hf, hf))
        cent_ref[:, b * blk:(b + 1) * blk] = hc
        nx_ref[:, b * blk:(b + 1) * blk] = hx
        ny_ref[:, b * blk:(b + 1) * blk] = hy
        nz_ref[:, b * blk:(b + 1) * blk] = hz


def _run_fps(xyz, npoint):
    B, N, _ = xyz.shape
    xt = jnp.transpose(xyz, (0, 2, 1))
    far0 = jax.random.randint(jax.random.key(1), (B,), 0, N)
    far0 = far0.astype(jnp.int32).reshape(B, 1)
    out_shapes = (
        jax.ShapeDtypeStruct((B, npoint), jnp.int32),
        jax.ShapeDtypeStruct((B, npoint), jnp.float32),
        jax.ShapeDtypeStruct((B, npoint), jnp.float32),
        jax.ShapeDtypeStruct((B, npoint), jnp.float32),
    )
    return pl.pallas_call(
        functools.partial(_fps_body, npoint=npoint),
        out_shape=out_shapes,
        scratch_shapes=[pltpu.VMEM((B, N), jnp.float32)],
    )(xt[:, 0], xt[:, 1], xt[:, 2], far0)


def _knn_body(nx_ref, ny_ref, nz_ref, x_ref, idx_ref, dist_ref, *, k):
    TS = nx_ref.shape[1]
    N = x_ref.shape[2]
    cx = nx_ref[0]
    cy = ny_ref[0]
    cz = nz_ref[0]
    pts = x_ref[0]
    px = pts[0:1]
    py = pts[1:2]
    pz = pts[2:3]
    s2 = cx * cx + cy * cy + cz * cz
    d2 = px * px + py * py + pz * pz
    q = jnp.concatenate([cx, cy, cz], axis=1)
    cross = jax.lax.dot_general(
        q.astype(jnp.bfloat16), pts.astype(jnp.bfloat16),
        (((1,), (0,)), ((), ())), preferred_element_type=jnp.float32)
    dist_ref[...] = jnp.maximum(s2 + d2 - 2.0 * cross, 0.0)
    col = lax.broadcasted_iota(jnp.int32, (TS, N), 1)
    kcol = lax.broadcasted_iota(jnp.int32, (TS, k), 1)

    def step(j, carry):
        a_prev, acc = carry
        d = jnp.where(col == a_prev, jnp.float32(1e30), dist_ref[...])
        dist_ref[...] = d
        m = jnp.min(d, axis=1, keepdims=True)
        a = jnp.min(jnp.where(d == m, col, jnp.int32(N)), axis=1,
                    keepdims=True)
        return a, acc + a * (kcol == j).astype(jnp.int32)

    a0 = jnp.full((TS, 1), N, jnp.int32)
    _, acc = lax.fori_loop(0, k, step, (a0, jnp.zeros((TS, k), jnp.int32)))
    idx_ref[0] = acc


def _run_knn(nx, ny, nz, xyz, k, ts):
    B, S = nx.shape
    N = xyz.shape[1]
    xt = jnp.transpose(xyz, (0, 2, 1))
    nx3 = nx.reshape(B, S, 1)
    ny3 = ny.reshape(B, S, 1)
    nz3 = nz.reshape(B, S, 1)
    grid = (B, S // ts)
    q_spec = pl.BlockSpec((1, ts, 1), lambda b, s: (b, s, 0))
    return pl.pallas_call(
        functools.partial(_knn_body, k=k),
        grid=grid,
        in_specs=[q_spec, q_spec, q_spec,
                  pl.BlockSpec((1, 3, N), lambda b, s: (b, 0, 0))],
        out_specs=pl.BlockSpec((1, ts, k), lambda b, s: (b, s, 0)),
        out_shape=jax.ShapeDtypeStruct((B, S, k), jnp.int32),
        scratch_shapes=[pltpu.VMEM((ts, N), jnp.float32)],
    )(nx3, ny3, nz3, xt)


def _run_sc_gather(tbl, gidx):
    P = gidx.shape[0]
    D = tbl.shape[1]
    info = plsc.get_sparse_core_info()
    NC, NS = info.num_cores, info.num_subcores
    NW = NC * NS
    per_w = P // NW
    CH = 128
    n_chunks = per_w // CH
    mesh = plsc.VectorSubcoreMesh(core_axis_name="c", subcore_axis_name="s")

    @functools.partial(
        pl.kernel, mesh=mesh,
        out_type=jax.ShapeDtypeStruct((P, D), jnp.float32),
        scratch_types=[
            pltpu.VMEM((CH,), jnp.int32),
            pltpu.VMEM((CH, D), jnp.float32),
            pltpu.SemaphoreType.DMA,
        ],
    )
    def gather_k(tbl_hbm, idx_hbm, out_hbm, idx_v, rows_v, sem):
        wid = lax.axis_index("s") * NC + lax.axis_index("c")

        def chunk(c, _):
            base = wid * per_w + c * CH
            pltpu.sync_copy(idx_hbm.at[pl.ds(base, CH)], idx_v)
            pltpu.async_copy(tbl_hbm.at[idx_v], rows_v, sem).wait()
            pltpu.sync_copy(rows_v, out_hbm.at[pl.ds(base, CH)])
            return 0

        lax.fori_loop(0, n_chunks, chunk, 0)

    return gather_k(tbl, gidx)


def _mlp_stats1_body(gr_ref, nx_ref, w1pt_ref, w1at_ref, st_ref, *, k):
    t = pl.program_id(0)
    g = gr_ref[...]
    TG = nx_ref.shape[0]
    y1 = jnp.dot(g, w1pt_ref[...], preferred_element_type=jnp.float32)
    nw = jnp.dot(nx_ref[...], w1at_ref[...],
                 preferred_element_type=jnp.float32)
    C1 = y1.shape[1]
    nwr = jnp.broadcast_to(nw[:, None, :], (TG, k, C1)).reshape(TG * k, C1)
    y1 = y1 - nwr
    s = jnp.sum(y1, axis=0, keepdims=True)
    q = jnp.sum(y1 * y1, axis=0, keepdims=True)
    contrib = jnp.concatenate(
        [s, q, jnp.zeros((6, C1), jnp.float32)], axis=0)

    @pl.when(t == 0)
    def _():
        st_ref[...] = jnp.zeros_like(st_ref)

    st_ref[...] += contrib


def _mlp_stats2_body(gr_ref, nx_ref, w1pt_ref, w1at_ref, sc1_ref, w2t_ref,
                     st_ref, *, k):
    t = pl.program_id(0)
    g = gr_ref[...]
    TG = nx_ref.shape[0]
    y1 = jnp.dot(g, w1pt_ref[...], preferred_element_type=jnp.float32)
    nw = jnp.dot(nx_ref[...], w1at_ref[...],
                 preferred_element_type=jnp.float32)
    C1 = y1.shape[1]
    nwr = jnp.broadcast_to(nw[:, None, :], (TG, k, C1)).reshape(TG * k, C1)
    h1 = jnp.maximum((y1 - nwr) * sc1_ref[0:1, :] + sc1_ref[1:2, :], 0.0)
    y2 = jnp.dot(h1, w2t_ref[...], preferred_element_type=jnp.float32)
    C2 = y2.shape[1]
    s = jnp.sum(y2, axis=0, keepdims=True)
    q = jnp.sum(y2 * y2, axis=0, keepdims=True)
    contrib = jnp.concatenate(
        [s, q, jnp.zeros((6, C2), jnp.float32)], axis=0)

    @pl.when(t == 0)
    def _():
        st_ref[...] = jnp.zeros_like(st_ref)

    st_ref[...] += contrib


def _mlp_final_body(gr_ref, nx_ref, w1pt_ref, w1at_ref, sc1_ref, w2t_ref,
                    sc2_ref, out_ref, *, k):
    g = gr_ref[...]
    TG = nx_ref.shape[0]
    y1 = jnp.dot(g, w1pt_ref[...], preferred_element_type=jnp.float32)
    nw = jnp.dot(nx_ref[...], w1at_ref[...],
                 preferred_element_type=jnp.float32)
    C1 = y1.shape[1]
    nwr = jnp.broadcast_to(nw[:, None, :], (TG, k, C1)).reshape(TG * k, C1)
    h1 = jnp.maximum((y1 - nwr) * sc1_ref[0:1, :] + sc1_ref[1:2, :], 0.0)
    y2 = jnp.dot(h1, w2t_ref[...], preferred_element_type=jnp.float32)
    C2 = y2.shape[1]
    o = jnp.maximum(y2 * sc2_ref[0:1, :] + sc2_ref[1:2, :], 0.0)
    out_ref[...] = jnp.max(o.reshape(TG, k, C2), axis=1)


def _run_mlp(gr, nxyz8, w1pt, w1at, w2t, g1, b1, g2, b2, k, tr):
    P, D = gr.shape
    C1 = w1pt.shape[1]
    C2 = w2t.shape[1]
    TG = tr // k
    n_t = P // tr
    grid = (n_t,)
    gr_spec = pl.BlockSpec((tr, D), lambda t: (t, 0))
    nx_spec = pl.BlockSpec((TG, 8), lambda t: (t, 0))
    w1pt_spec = pl.BlockSpec(w1pt.shape, lambda t: (0, 0))
    w1at_spec = pl.BlockSpec(w1at.shape, lambda t: (0, 0))
    w2t_spec = pl.BlockSpec(w2t.shape, lambda t: (0, 0))
    st_spec = pl.BlockSpec((8, C1), lambda t: (0, 0))

    st1 = pl.pallas_call(
        functools.partial(_mlp_stats1_body, k=k),
        grid=grid,
        in_specs=[gr_spec, nx_spec, w1pt_spec, w1at_spec],
        out_specs=st_spec,
        out_shape=jax.ShapeDtypeStruct((8, C1), jnp.float32),
    )(gr, nxyz8, w1pt, w1at)

    n = jnp.float32(P)
    mean1 = st1[0] / n
    var1 = st1[1] / n - mean1 * mean1
    scale1 = g1 / jnp.sqrt(var1 + 1e-5)
    shift1 = b1 - mean1 * scale1
    sc1 = jnp.stack([scale1, shift1], axis=0)

    st2_spec = pl.BlockSpec((8, C2), lambda t: (0, 0))
    sc1_spec = pl.BlockSpec((2, C1), lambda t: (0, 0))
    st2 = pl.pallas_call(
        functools.partial(_mlp_stats2_body, k=k),
        grid=grid,
        in_specs=[gr_spec, nx_spec, w1pt_spec, w1at_spec, sc1_spec, w2t_spec],
        out_specs=st2_spec,
        out_shape=jax.ShapeDtypeStruct((8, C2), jnp.float32),
    )(gr, nxyz8, w1pt, w1at, sc1, w2t)

    mean2 = st2[0] / n
    var2 = st2[1] / n - mean2 * mean2
    scale2 = g2 / jnp.sqrt(var2 + 1e-5)
    shift2 = b2 - mean2 * scale2
    sc2 = jnp.stack([scale2, shift2], axis=0)

    sc2_spec = pl.BlockSpec((2, C2), lambda t: (0, 0))
    out = pl.pallas_call(
        functools.partial(_mlp_final_body, k=k),
        grid=grid,
        in_specs=[gr_spec, nx_spec, w1pt_spec, w1at_spec, sc1_spec, w2t_spec,
                  sc2_spec],
        out_specs=pl.BlockSpec((TG, C2), lambda t: (t, 0)),
        out_shape=jax.ShapeDtypeStruct((P // k, C2), jnp.float32),
    )(gr, nxyz8, w1pt, w1at, sc1, w2t, sc2)
    return out


def kernel(xyz, features, W1, g1, b1, W2, g2, b2):
    B, N, _ = xyz.shape
    C = features.shape[1]
    S = NPOINT
    cent, nx, ny, nz = _run_fps(xyz, S)
    gidx = _run_knn(nx, ny, nz, xyz, K, min(512, S))

    Dpad = 128
    feat_t = jnp.transpose(features, (0, 2, 1))
    tbl = jnp.concatenate(
        [feat_t, xyz, jnp.zeros((B, N, Dpad - C - 3), jnp.float32)],
        axis=-1).reshape(B * N, Dpad)
    flat_idx = (gidx + (jnp.arange(B, dtype=jnp.int32) * N)[:, None, None]
                ).reshape(B * S * K)
    gr = _run_sc_gather(tbl, flat_idx)

    C1 = W1.shape[0]
    C2 = W2.shape[0]
    w1pt = jnp.zeros((Dpad, C1), jnp.float32)
    w1pt = w1pt.at[:C].set(W1[:, 3:].T)
    w1pt = w1pt.at[C:C + 3].set(W1[:, :3].T)
    w1at = jnp.zeros((8, C1), jnp.float32).at[:3].set(W1[:, :3].T)
    w2t = W2.T
    nxyz8 = jnp.zeros((B * S, 8), jnp.float32)
    nxyz8 = nxyz8.at[:, 0].set(nx.reshape(-1))
    nxyz8 = nxyz8.at[:, 1].set(ny.reshape(-1))
    nxyz8 = nxyz8.at[:, 2].set(nz.reshape(-1))

    pooled = _run_mlp(gr, nxyz8, w1pt, w1at, w2t, g1, b1, g2, b2, K,
                      min(4096, B * S * K))

    new_xyz = jnp.stack([nx, ny, nz], axis=-1)
    new_features = pooled.reshape(B, S, C2).transpose(0, 2, 1)
    return (new_xyz, new_features)
